# baseline (speedup 1.0000x reference)
import functools

import numpy as np
import jax
import jax.numpy as jnp
from jax import lax
from jax.experimental import pallas as pl
from jax.experimental.pallas import tpu as pltpu
from jax.experimental.pallas import tpu_sc as plsc

N = 50000
E = 800000
HID = 64
T = 5
PACK = 4
BM = 2000
M4 = E // PACK

NC = 2
NS = 16
CH = 128
QROWS = CH // PACK
NCHUNK = E // CH
GRP = 16
NGRP_FULL = 24
EI_ROWS = NCHUNK + GRP
N_PAD = 50048
RPT = N_PAD // NS
HALF = 32


def _block_diag(w, c):
    k, n = w.shape
    out = jnp.zeros((c * k, c * n), w.dtype)
    for i in range(c):
        out = out.at[i * k:(i + 1) * k, i * n:(i + 1) * n].set(w)
    return out


def _edge_mlp_body(ea_ref, w1_ref, b1_ref, w2_ref, b2_ref, w3_ref, b3_ref,
                   lo_ref, hi_ref):
    h = jnp.dot(ea_ref[...], w1_ref[...], preferred_element_type=jnp.float32)
    h = jax.nn.relu(h + b1_ref[...])
    h = jnp.dot(h, w2_ref[...], preferred_element_type=jnp.float32)
    h = jax.nn.relu(h + b2_ref[...])
    h = jnp.dot(h, w3_ref[...], preferred_element_type=jnp.float32)
    h = (h + b3_ref[...]).astype(jnp.bfloat16)
    lo_ref[...] = h[:, :128]
    hi_ref[...] = h[:, 128:]


def _edge_mlp(ea4, w1b, b1b, w2b, b2b, w3b, b3b):
    m = ea4.shape[0]
    kdim = ea4.shape[1]
    nd = w1b.shape[1]
    grid = m // BM
    return pl.pallas_call(
        _edge_mlp_body,
        grid=(grid,),
        in_specs=[
            pl.BlockSpec((BM, kdim), lambda i: (i, 0)),
            pl.BlockSpec((kdim, nd), lambda i: (0, 0)),
            pl.BlockSpec((1, nd), lambda i: (0, 0)),
            pl.BlockSpec((nd, nd), lambda i: (0, 0)),
            pl.BlockSpec((1, nd), lambda i: (0, 0)),
            pl.BlockSpec((nd, nd), lambda i: (0, 0)),
            pl.BlockSpec((1, nd), lambda i: (0, 0)),
        ],
        out_specs=[
            pl.BlockSpec((BM, 128), lambda i: (i, 0)),
            pl.BlockSpec((BM, 128), lambda i: (i, 0)),
        ],
        out_shape=[
            jax.ShapeDtypeStruct((M4, 128), jnp.bfloat16),
            jax.ShapeDtypeStruct((M4, 128), jnp.bfloat16),
        ],
    )(ea4, w1b, b1b, w2b, b2b, w3b, b3b)


_SC_MESH = plsc.VectorSubcoreMesh(core_axis_name="c", subcore_axis_name="s")


@functools.partial(
    pl.kernel,
    out_type=jax.ShapeDtypeStruct((NC * N_PAD, HALF), jnp.float32),
    mesh=_SC_MESH,
    scratch_types=[
        pltpu.VMEM((GRP, 128), jnp.int32),
        pltpu.VMEM((GRP, 128), jnp.int32),
        pltpu.VMEM((2, CH, HALF), jnp.bfloat16),
        pltpu.VMEM((2, QROWS, 128), jnp.bfloat16),
        pltpu.VMEM((2, CH, HALF), jnp.float32),
        pltpu.VMEM_SHARED((N_PAD, HALF), jnp.float32),
        pltpu.SemaphoreType.DMA,
        pltpu.SemaphoreType.DMA,
    ],
    compiler_params=pltpu.CompilerParams(use_tc_tiling_on_sc=False,
                                         needs_layout_passes=False),
)
def _sc_round(ei3_hbm, klo_hbm, khi_hbm, vlo_hbm, vhi_hbm, out_hbm,
              sgrp, dgrp, vbuf, kbuf, msg, acc, gsem, ksem):
    c = lax.axis_index("c")
    s = lax.axis_index("s")

    zeros16 = jnp.zeros((16,), jnp.float32)

    def zb(i, carry):
        msg[0, i, pl.ds(0, 16)] = zeros16
        msg[0, i, pl.ds(16, 16)] = zeros16
        return carry

    lax.fori_loop(0, CH, zb, 0)
    r0 = s * RPT
    for z in range(RPT // CH):
        pltpu.sync_copy(msg.at[0], acc.at[pl.ds(r0 + z * CH, CH)])
    pltpu.sync_copy(msg.at[0, pl.ds(0, RPT % CH)],
                    acc.at[pl.ds(r0 + (RPT // CH) * CH, RPT % CH)])
    plsc.subcore_barrier()

    nchunks = 390 + (s < 10).astype(jnp.int32)
    c0 = s * 390 + jnp.minimum(s, 10)

    def run(kap_hbm, v_hbm, obase):
        def fire(cid, b):
            pltpu.async_copy(kap_hbm.at[pl.ds(cid * QROWS, QROWS)],
                             kbuf.at[b], ksem)

        def mul(b):
            @plsc.parallel_loop(0, QROWS, unroll=2)
            def _(q):
                for be in range(PACK):
                    e = q * PACK + be
                    ka, kb2 = plsc.unpack(kbuf[b, q, pl.ds(be * 32, 32)],
                                          format=plsc.PackFormat.INTERLEAVED)
                    va, vb2 = plsc.unpack(vbuf[b, e, pl.ds(0, 32)],
                                          format=plsc.PackFormat.INTERLEAVED)
                    msg[b, e, pl.ds(0, 16)] = ka * va
                    msg[b, e, pl.ds(16, 16)] = kb2 * vb2

        def group_body(g, carry):
            base = c0 + g * GRP
            pltpu.sync_copy(ei3_hbm.at[0, pl.ds(base, GRP)], sgrp)
            pltpu.sync_copy(ei3_hbm.at[1, pl.ds(base, GRP)], dgrp)
            pltpu.async_copy(v_hbm.at[sgrp.at[0]], vbuf.at[0], gsem)
            fire(base, 0)
            for t in range(GRP):
                b = t % 2
                nb = 1 - b
                if t + 1 < GRP:
                    pltpu.async_copy(v_hbm.at[sgrp.at[t + 1]], vbuf.at[nb],
                                     gsem)
                    fire(base + t + 1, nb)
                pltpu.make_async_copy(kap_hbm.at[pl.ds(0, QROWS)],
                                      kbuf.at[b], ksem).wait()
                pltpu.make_async_copy(v_hbm.at[sgrp.at[t]],
                                      vbuf.at[b], gsem).wait()
                mul(b)
                pltpu.sync_copy(msg.at[b], acc.at[dgrp.at[t]], add=True)
            return carry

        lax.fori_loop(0, NGRP_FULL, group_body, 0)

        def tail_body(i, carry):
            cid = c0 + i
            pltpu.sync_copy(ei3_hbm.at[0, pl.ds(cid, 1)],
                            sgrp.at[pl.ds(0, 1)])
            pltpu.sync_copy(ei3_hbm.at[1, pl.ds(cid, 1)],
                            dgrp.at[pl.ds(0, 1)])
            pltpu.async_copy(v_hbm.at[sgrp.at[0]], vbuf.at[0], gsem)
            fire(cid, 0)
            pltpu.make_async_copy(kap_hbm.at[pl.ds(0, QROWS)],
                                  kbuf.at[0], ksem).wait()
            pltpu.make_async_copy(v_hbm.at[sgrp.at[0]],
                                  vbuf.at[0], gsem).wait()
            mul(0)
            pltpu.sync_copy(msg.at[0], acc.at[dgrp.at[0]], add=True)
            return carry

        lax.fori_loop(NGRP_FULL * GRP, nchunks, tail_body, 0)

        plsc.subcore_barrier()
        pltpu.sync_copy(acc.at[pl.ds(r0, RPT)],
                        out_hbm.at[pl.ds(obase + r0, RPT)])

    @pl.when(c == 0)
    def _():
        run(klo_hbm, vlo_hbm, 0)

    @pl.when(c == 1)
    def _():
        run(khi_hbm, vhi_hbm, N_PAD)


_ILV = np.arange(32).reshape(2, 16).T.reshape(-1)
_PFULL = np.concatenate([_ILV, 32 + _ILV])
_PM = np.eye(64, dtype=np.float32)[_PFULL].T
_KP_BASE = np.arange(PACK)[:, None] * HID
_KPERM = np.concatenate(
    [(_KP_BASE + hh * HALF + _ILV[None, :]).reshape(-1) for hh in range(2)])
_PK = np.eye(PACK * HID, dtype=np.float32)[_KPERM].T


def kernel(x, edge_index, edge_attr, lift_W1, lift_b1, lift_W2, lift_b2,
           kW1, kb1, kW2, kb2, kW3, kb3, sW, sb, pW1, pb1, pW2, pb2):
    dst = edge_index[1]
    ei3 = jnp.pad(edge_index.reshape(2, NCHUNK, 128),
                  ((0, 0), (0, EI_ROWS - NCHUNK), (0, 0)))
    mu = jnp.mean(edge_attr, axis=0, keepdims=True)
    sd = jnp.std(edge_attr, axis=0, keepdims=True)
    ea = (edge_attr - mu) / (sd + 1e-6)
    ea4 = ea.reshape(E // PACK, PACK * ea.shape[1])

    pm = jnp.asarray(_PM)
    pk = jnp.asarray(_PK)
    v = jax.nn.relu(x @ lift_W1 + lift_b1) @ (lift_W2 @ pm) + lift_b2 @ pm
    deg = jax.ops.segment_sum(jnp.ones((E,), jnp.float32), dst, num_segments=N)
    inv_deg = 1.0 / jnp.maximum(deg, 1.0)[:, None]

    for t in range(T):
        w1b = _block_diag(kW1[t], PACK)
        b1b = jnp.tile(kb1[t], PACK)[None, :]
        w2b = _block_diag(kW2[t], PACK)
        b2b = jnp.tile(kb2[t], PACK)[None, :]
        w3b = _block_diag(kW3[t], PACK) @ pk
        b3b = (jnp.tile(kb3[t], PACK) @ pk)[None, :]
        klo, khi = _edge_mlp(ea4, w1b, b1b, w2b, b2b, w3b, b3b)
        vlo = v[:, :HALF].astype(jnp.bfloat16)
        vhi = v[:, HALF:].astype(jnp.bfloat16)
        agg2 = _sc_round(ei3, klo, khi, vlo, vhi)
        agg = jnp.concatenate([agg2[:N], agg2[N_PAD:N_PAD + N]], axis=1) * inv_deg
        v = jax.nn.relu(v @ (pm.T @ sW[t] @ pm) + sb[t] @ pm + agg @ pm)

    out = jax.nn.relu(v @ (pm.T @ pW1) + pb1) @ pW2 + pb2
    return out


# speedup vs baseline: 1.1141x; 1.1141x over previous
import functools

import numpy as np
import jax
import jax.numpy as jnp
from jax import lax
from jax.experimental import pallas as pl
from jax.experimental.pallas import tpu as pltpu
from jax.experimental.pallas import tpu_sc as plsc

N = 50000
E = 800000
HID = 64
T = 5
PACK = 4
BM = 2000
M4 = E // PACK

NC = 2
NS = 16
CH = 128
QROWS = CH // PACK
NCHUNK = E // CH
GRP = 8
NGRP_FULL = 48
EI_ROWS = NCHUNK + GRP
N_PAD = 50048
RPT = N_PAD // NS
HALF = 32


def _block_diag(w, c):
    k, n = w.shape
    out = jnp.zeros((c * k, c * n), w.dtype)
    for i in range(c):
        out = out.at[i * k:(i + 1) * k, i * n:(i + 1) * n].set(w)
    return out


def _edge_mlp_body(ea_ref, w1_ref, b1_ref, w2_ref, b2_ref, w3_ref, b3_ref,
                   lo_ref, hi_ref):
    h = jnp.dot(ea_ref[...], w1_ref[...], preferred_element_type=jnp.float32)
    h = jax.nn.relu(h + b1_ref[...])
    h = jnp.dot(h, w2_ref[...], preferred_element_type=jnp.float32)
    h = jax.nn.relu(h + b2_ref[...])
    h = jnp.dot(h, w3_ref[...], preferred_element_type=jnp.float32)
    h = h + b3_ref[...]
    lo_ref[...] = h[:, :128]
    hi_ref[...] = h[:, 128:]


def _edge_mlp(ea4, w1b, b1b, w2b, b2b, w3b, b3b):
    m = ea4.shape[0]
    kdim = ea4.shape[1]
    nd = w1b.shape[1]
    grid = m // BM
    return pl.pallas_call(
        _edge_mlp_body,
        grid=(grid,),
        in_specs=[
            pl.BlockSpec((BM, kdim), lambda i: (i, 0)),
            pl.BlockSpec((kdim, nd), lambda i: (0, 0)),
            pl.BlockSpec((1, nd), lambda i: (0, 0)),
            pl.BlockSpec((nd, nd), lambda i: (0, 0)),
            pl.BlockSpec((1, nd), lambda i: (0, 0)),
            pl.BlockSpec((nd, nd), lambda i: (0, 0)),
            pl.BlockSpec((1, nd), lambda i: (0, 0)),
        ],
        out_specs=[
            pl.BlockSpec((BM, 128), lambda i: (i, 0)),
            pl.BlockSpec((BM, 128), lambda i: (i, 0)),
        ],
        out_shape=[
            jax.ShapeDtypeStruct((M4, 128), jnp.float32),
            jax.ShapeDtypeStruct((M4, 128), jnp.float32),
        ],
    )(ea4, w1b, b1b, w2b, b2b, w3b, b3b)


_SC_MESH = plsc.VectorSubcoreMesh(core_axis_name="c", subcore_axis_name="s")


@functools.partial(
    pl.kernel,
    out_type=jax.ShapeDtypeStruct((NC * N_PAD, HALF), jnp.float32),
    mesh=_SC_MESH,
    scratch_types=[
        pltpu.VMEM((GRP, 128), jnp.int32),
        pltpu.VMEM((GRP, 128), jnp.int32),
        pltpu.VMEM((2, CH, HALF), jnp.bfloat16),
        pltpu.VMEM((2, QROWS, 128), jnp.float32),
        pltpu.VMEM((2, CH, HALF), jnp.float32),
        pltpu.VMEM_SHARED((N_PAD, HALF), jnp.float32),
        pltpu.SemaphoreType.DMA,
        pltpu.SemaphoreType.DMA,
    ],
    compiler_params=pltpu.CompilerParams(use_tc_tiling_on_sc=False,
                                         needs_layout_passes=False),
)
def _sc_round(ei3_hbm, klo_hbm, khi_hbm, vlo_hbm, vhi_hbm, out_hbm,
              sgrp, dgrp, vbuf, kbuf, msg, acc, gsem, ksem):
    c = lax.axis_index("c")
    s = lax.axis_index("s")

    zeros16 = jnp.zeros((16,), jnp.float32)

    def zb(i, carry):
        msg[0, i, pl.ds(0, 16)] = zeros16
        msg[0, i, pl.ds(16, 16)] = zeros16
        return carry

    lax.fori_loop(0, CH, zb, 0)
    r0 = s * RPT
    for z in range(RPT // CH):
        pltpu.sync_copy(msg.at[0], acc.at[pl.ds(r0 + z * CH, CH)])
    pltpu.sync_copy(msg.at[0, pl.ds(0, RPT % CH)],
                    acc.at[pl.ds(r0 + (RPT // CH) * CH, RPT % CH)])
    plsc.subcore_barrier()

    nchunks = 390 + (s < 10).astype(jnp.int32)
    c0 = s * 390 + jnp.minimum(s, 10)

    def run(kap_hbm, v_hbm, obase):
        def fire(cid, b):
            pltpu.async_copy(kap_hbm.at[pl.ds(cid * QROWS, QROWS)],
                             kbuf.at[b], ksem)

        def mul(b):
            @plsc.parallel_loop(0, QROWS, unroll=2)
            def _(q):
                for be in range(PACK):
                    e = q * PACK + be
                    va, vb2 = plsc.unpack(vbuf[b, e, pl.ds(0, 32)],
                                          format=plsc.PackFormat.INTERLEAVED)
                    msg[b, e, pl.ds(0, 16)] = (
                        kbuf[b, q, pl.ds(be * 32, 16)] * va)
                    msg[b, e, pl.ds(16, 16)] = (
                        kbuf[b, q, pl.ds(be * 32 + 16, 16)] * vb2)

        def group_body(g, carry):
            base = c0 + g * GRP
            pltpu.sync_copy(ei3_hbm.at[0, pl.ds(base, GRP)], sgrp)
            pltpu.sync_copy(ei3_hbm.at[1, pl.ds(base, GRP)], dgrp)
            pltpu.async_copy(v_hbm.at[sgrp.at[0]], vbuf.at[0], gsem)
            fire(base, 0)
            for t in range(GRP):
                b = t % 2
                nb = 1 - b
                if t + 1 < GRP:
                    pltpu.async_copy(v_hbm.at[sgrp.at[t + 1]], vbuf.at[nb],
                                     gsem)
                    fire(base + t + 1, nb)
                pltpu.make_async_copy(kap_hbm.at[pl.ds(0, QROWS)],
                                      kbuf.at[b], ksem).wait()
                pltpu.make_async_copy(v_hbm.at[sgrp.at[t]],
                                      vbuf.at[b], gsem).wait()
                mul(b)
                pltpu.sync_copy(msg.at[b], acc.at[dgrp.at[t]], add=True)
            return carry

        lax.fori_loop(0, NGRP_FULL, group_body, 0)

        def tail_body(i, carry):
            cid = c0 + i
            pltpu.sync_copy(ei3_hbm.at[0, pl.ds(cid, 1)],
                            sgrp.at[pl.ds(0, 1)])
            pltpu.sync_copy(ei3_hbm.at[1, pl.ds(cid, 1)],
                            dgrp.at[pl.ds(0, 1)])
            pltpu.async_copy(v_hbm.at[sgrp.at[0]], vbuf.at[0], gsem)
            fire(cid, 0)
            pltpu.make_async_copy(kap_hbm.at[pl.ds(0, QROWS)],
                                  kbuf.at[0], ksem).wait()
            pltpu.make_async_copy(v_hbm.at[sgrp.at[0]],
                                  vbuf.at[0], gsem).wait()
            mul(0)
            pltpu.sync_copy(msg.at[0], acc.at[dgrp.at[0]], add=True)
            return carry

        lax.fori_loop(NGRP_FULL * GRP, nchunks, tail_body, 0)

        plsc.subcore_barrier()
        pltpu.sync_copy(acc.at[pl.ds(r0, RPT)],
                        out_hbm.at[pl.ds(obase + r0, RPT)])

    @pl.when(c == 0)
    def _():
        run(klo_hbm, vlo_hbm, 0)

    @pl.when(c == 1)
    def _():
        run(khi_hbm, vhi_hbm, N_PAD)


_ILV = np.arange(32).reshape(2, 16).T.reshape(-1)
_PFULL = np.concatenate([_ILV, 32 + _ILV])
_PM = np.eye(64, dtype=np.float32)[_PFULL].T
_KP_BASE = np.arange(PACK)[:, None] * HID
_KPERM = np.concatenate(
    [(_KP_BASE + hh * HALF + np.arange(HALF)[None, :]).reshape(-1)
     for hh in range(2)])
_PK = np.eye(PACK * HID, dtype=np.float32)[_KPERM].T


def kernel(x, edge_index, edge_attr, lift_W1, lift_b1, lift_W2, lift_b2,
           kW1, kb1, kW2, kb2, kW3, kb3, sW, sb, pW1, pb1, pW2, pb2):
    dst = edge_index[1]
    ei3 = jnp.pad(edge_index.reshape(2, NCHUNK, 128),
                  ((0, 0), (0, EI_ROWS - NCHUNK), (0, 0)))
    mu = jnp.mean(edge_attr, axis=0, keepdims=True)
    sd = jnp.std(edge_attr, axis=0, keepdims=True)
    ea = (edge_attr - mu) / (sd + 1e-6)
    ea4 = ea.reshape(E // PACK, PACK * ea.shape[1])

    pm = jnp.asarray(_PM)
    pk = jnp.asarray(_PK)
    v = jax.nn.relu(x @ lift_W1 + lift_b1) @ (lift_W2 @ pm) + lift_b2 @ pm
    deg = jax.ops.segment_sum(jnp.ones((E,), jnp.float32), dst, num_segments=N)
    inv_deg = 1.0 / jnp.maximum(deg, 1.0)[:, None]

    for t in range(T):
        w1b = _block_diag(kW1[t], PACK)
        b1b = jnp.tile(kb1[t], PACK)[None, :]
        w2b = _block_diag(kW2[t], PACK)
        b2b = jnp.tile(kb2[t], PACK)[None, :]
        w3b = _block_diag(kW3[t], PACK) @ pk
        b3b = (jnp.tile(kb3[t], PACK) @ pk)[None, :]
        klo, khi = _edge_mlp(ea4, w1b, b1b, w2b, b2b, w3b, b3b)
        vlo = v[:, :HALF].astype(jnp.bfloat16)
        vhi = v[:, HALF:].astype(jnp.bfloat16)
        agg2 = _sc_round(ei3, klo, khi, vlo, vhi)
        agg = jnp.concatenate([agg2[:N], agg2[N_PAD:N_PAD + N]], axis=1) * inv_deg
        v = jax.nn.relu(v @ (pm.T @ sW[t] @ pm) + sb[t] @ pm + agg @ pm)

    out = jax.nn.relu(v @ (pm.T @ pW1) + pb1) @ pW2 + pb2
    return out


# speedup vs baseline: 1.3657x; 1.2258x over previous
import functools

import numpy as np
import jax
import jax.numpy as jnp
from jax import lax
from jax.experimental import pallas as pl
from jax.experimental.pallas import tpu as pltpu
from jax.experimental.pallas import tpu_sc as plsc

N = 50000
E = 800000
HID = 64
T = 5
PACK = 4
BM = 2000
M4 = E // PACK

NC = 2
NS = 16
CH = 128
QROWS = CH // PACK
NCHUNK = E // CH
GRP = 8
NGRP_FULL = 48
EI_ROWS = NCHUNK + GRP
N_PAD = 50048
RPT = N_PAD // NS
HALF = 32


def _block_diag(w, c):
    k, n = w.shape
    out = jnp.zeros((c * k, c * n), w.dtype)
    for i in range(c):
        out = out.at[i * k:(i + 1) * k, i * n:(i + 1) * n].set(w)
    return out


def _edge_mlp_body(ea_ref, w1_ref, b1_ref, w2_ref, b2_ref, w3_ref, b3_ref,
                   lo_ref, hi_ref):
    h = jnp.dot(ea_ref[...], w1_ref[...], preferred_element_type=jnp.float32)
    h = jax.nn.relu(h + b1_ref[...])
    h = jnp.dot(h, w2_ref[...], preferred_element_type=jnp.float32)
    h = jax.nn.relu(h + b2_ref[...])
    h = jnp.dot(h, w3_ref[...], preferred_element_type=jnp.float32)
    h = h + b3_ref[...]
    lo_ref[...] = h[:, :128]
    hi_ref[...] = h[:, 128:]


def _edge_mlp(ea4, w1b, b1b, w2b, b2b, w3b, b3b):
    m = ea4.shape[0]
    kdim = ea4.shape[1]
    nd = w1b.shape[1]
    grid = m // BM
    return pl.pallas_call(
        _edge_mlp_body,
        grid=(grid,),
        in_specs=[
            pl.BlockSpec((BM, kdim), lambda i: (i, 0)),
            pl.BlockSpec((kdim, nd), lambda i: (0, 0)),
            pl.BlockSpec((1, nd), lambda i: (0, 0)),
            pl.BlockSpec((nd, nd), lambda i: (0, 0)),
            pl.BlockSpec((1, nd), lambda i: (0, 0)),
            pl.BlockSpec((nd, nd), lambda i: (0, 0)),
            pl.BlockSpec((1, nd), lambda i: (0, 0)),
        ],
        out_specs=[
            pl.BlockSpec((BM, 128), lambda i: (i, 0)),
            pl.BlockSpec((BM, 128), lambda i: (i, 0)),
        ],
        out_shape=[
            jax.ShapeDtypeStruct((M4, 128), jnp.float32),
            jax.ShapeDtypeStruct((M4, 128), jnp.float32),
        ],
    )(ea4, w1b, b1b, w2b, b2b, w3b, b3b)


_SC_MESH = plsc.VectorSubcoreMesh(core_axis_name="c", subcore_axis_name="s")


@functools.partial(
    pl.kernel,
    out_type=jax.ShapeDtypeStruct((NC * N_PAD, HALF), jnp.float32),
    mesh=_SC_MESH,
    scratch_types=[
        pltpu.VMEM((GRP, 128), jnp.int32),
        pltpu.VMEM((GRP, 128), jnp.int32),
        pltpu.VMEM((2, CH, HALF), jnp.bfloat16),
        pltpu.VMEM((2, QROWS, 128), jnp.float32),
        pltpu.VMEM((2, CH, HALF), jnp.float32),
        pltpu.VMEM_SHARED((N_PAD, HALF), jnp.float32),
        pltpu.SemaphoreType.DMA,
        pltpu.SemaphoreType.DMA,
    ],
    compiler_params=pltpu.CompilerParams(use_tc_tiling_on_sc=False,
                                         needs_layout_passes=False),
)
def _sc_round(ei3_hbm, klo_hbm, khi_hbm, vlo_hbm, vhi_hbm, out_hbm,
              sgrp, dgrp, vbuf, kbuf, msg, acc, gsem, ksem):
    c = lax.axis_index("c")
    s = lax.axis_index("s")

    zeros16 = jnp.zeros((16,), jnp.float32)

    def zb(i, carry):
        msg[0, i, pl.ds(0, 16)] = zeros16
        msg[0, i, pl.ds(16, 16)] = zeros16
        return carry

    lax.fori_loop(0, CH, zb, 0)
    r0 = s * RPT
    for z in range(RPT // CH):
        pltpu.sync_copy(msg.at[0], acc.at[pl.ds(r0 + z * CH, CH)])
    pltpu.sync_copy(msg.at[0, pl.ds(0, RPT % CH)],
                    acc.at[pl.ds(r0 + (RPT // CH) * CH, RPT % CH)])
    plsc.subcore_barrier()

    nchunks = 390 + (s < 10).astype(jnp.int32)
    c0 = s * 390 + jnp.minimum(s, 10)

    def run(kap_hbm, v_hbm, obase):
        def fire(cid, b):
            pltpu.async_copy(kap_hbm.at[pl.ds(cid * QROWS, QROWS)],
                             kbuf.at[b], ksem)

        def mul(b):
            @plsc.parallel_loop(0, QROWS, unroll=2)
            def _(q):
                for be in range(PACK):
                    e = q * PACK + be
                    va, vb2 = plsc.unpack(vbuf[b, e, pl.ds(0, 32)],
                                          format=plsc.PackFormat.INTERLEAVED)
                    msg[b, e, pl.ds(0, 16)] = (
                        kbuf[b, q, pl.ds(be * 32, 16)] * va)
                    msg[b, e, pl.ds(16, 16)] = (
                        kbuf[b, q, pl.ds(be * 32 + 16, 16)] * vb2)

        def group_body(g, carry):
            base = c0 + g * GRP
            pltpu.sync_copy(ei3_hbm.at[0, pl.ds(base, GRP)], sgrp)
            pltpu.sync_copy(ei3_hbm.at[1, pl.ds(base, GRP)], dgrp)
            pltpu.async_copy(v_hbm.at[sgrp.at[0]], vbuf.at[0], gsem)
            fire(base, 0)
            for t in range(GRP):
                b = t % 2
                nb = 1 - b
                if t + 1 < GRP:
                    pltpu.async_copy(v_hbm.at[sgrp.at[t + 1]], vbuf.at[nb],
                                     gsem)
                    fire(base + t + 1, nb)
                pltpu.make_async_copy(kap_hbm.at[pl.ds(0, QROWS)],
                                      kbuf.at[b], ksem).wait()
                pltpu.make_async_copy(v_hbm.at[sgrp.at[t]],
                                      vbuf.at[b], gsem).wait()
                mul(b)
                pltpu.sync_copy(msg.at[b], acc.at[dgrp.at[t]], add=True)
            return carry

        lax.fori_loop(0, NGRP_FULL, group_body, 0)

        def tail_body(i, carry):
            cid = c0 + i
            pltpu.sync_copy(ei3_hbm.at[0, pl.ds(cid, 1)],
                            sgrp.at[pl.ds(0, 1)])
            pltpu.sync_copy(ei3_hbm.at[1, pl.ds(cid, 1)],
                            dgrp.at[pl.ds(0, 1)])
            pltpu.async_copy(v_hbm.at[sgrp.at[0]], vbuf.at[0], gsem)
            fire(cid, 0)
            pltpu.make_async_copy(kap_hbm.at[pl.ds(0, QROWS)],
                                  kbuf.at[0], ksem).wait()
            pltpu.make_async_copy(v_hbm.at[sgrp.at[0]],
                                  vbuf.at[0], gsem).wait()
            mul(0)
            pltpu.sync_copy(msg.at[0], acc.at[dgrp.at[0]], add=True)
            return carry

        lax.fori_loop(NGRP_FULL * GRP, nchunks, tail_body, 0)

        plsc.subcore_barrier()
        pltpu.sync_copy(acc.at[pl.ds(r0, RPT)],
                        out_hbm.at[pl.ds(obase + r0, RPT)])

    @pl.when(c == 0)
    def _():
        run(klo_hbm, vlo_hbm, 0)

    @pl.when(c == 1)
    def _():
        run(khi_hbm, vhi_hbm, N_PAD)


_ILV = np.arange(32).reshape(2, 16).T.reshape(-1)
_PFULL = np.concatenate([_ILV, 32 + _ILV])
_PM = np.eye(64, dtype=np.float32)[_PFULL].T
_KP_BASE = np.arange(PACK)[:, None] * HID
_KPERM = np.concatenate(
    [(_KP_BASE + hh * HALF + np.arange(HALF)[None, :]).reshape(-1)
     for hh in range(2)])
_PK = np.eye(PACK * HID, dtype=np.float32)[_KPERM].T


def kernel(x, edge_index, edge_attr, lift_W1, lift_b1, lift_W2, lift_b2,
           kW1, kb1, kW2, kb2, kW3, kb3, sW, sb, pW1, pb1, pW2, pb2):
    ei3 = jnp.pad(edge_index.reshape(2, NCHUNK, 128),
                  ((0, 0), (0, EI_ROWS - NCHUNK), (0, 0)))
    mu = jnp.mean(edge_attr, axis=0, keepdims=True)
    sd = jnp.std(edge_attr, axis=0, keepdims=True)
    ea = (edge_attr - mu) / (sd + 1e-6)
    ea4 = ea.reshape(E // PACK, PACK * ea.shape[1])

    pm = jnp.asarray(_PM)
    pk = jnp.asarray(_PK)
    v = jax.nn.relu(x @ lift_W1 + lift_b1) @ (lift_W2 @ pm) + lift_b2 @ pm
    kones = jnp.ones((M4, 128), jnp.float32)
    vones = jnp.ones((N, HALF), jnp.bfloat16)
    deg = _sc_round(ei3, kones, kones, vones, vones)[:N, 0]
    inv_deg = 1.0 / jnp.maximum(deg, 1.0)[:, None]

    for t in range(T):
        w1b = _block_diag(kW1[t], PACK)
        b1b = jnp.tile(kb1[t], PACK)[None, :]
        w2b = _block_diag(kW2[t], PACK)
        b2b = jnp.tile(kb2[t], PACK)[None, :]
        w3b = _block_diag(kW3[t], PACK) @ pk
        b3b = (jnp.tile(kb3[t], PACK) @ pk)[None, :]
        klo, khi = _edge_mlp(ea4, w1b, b1b, w2b, b2b, w3b, b3b)
        vlo = v[:, :HALF].astype(jnp.bfloat16)
        vhi = v[:, HALF:].astype(jnp.bfloat16)
        agg2 = _sc_round(ei3, klo, khi, vlo, vhi)
        agg = jnp.concatenate([agg2[:N], agg2[N_PAD:N_PAD + N]], axis=1) * inv_deg
        v = jax.nn.relu(v @ (pm.T @ sW[t] @ pm) + sb[t] @ pm + agg @ pm)

    out = jax.nn.relu(v @ (pm.T @ pW1) + pb1) @ pW2 + pb2
    return out
